# baseline (speedup 1.0000x reference)
import functools

import jax
import jax.numpy as jnp
from jax import lax
from jax.experimental import pallas as pl
from jax.experimental.pallas import tpu as pltpu
from jax.experimental.pallas import tpu_sc as plsc

_IDX_CHUNK = 128
_PACK = 512


def _pack_body(eye_ref, in_ref, out_ref):
    x = in_ref[...]
    eye = eye_ref[...]
    for m in range(_PACK // 512):
        c0 = m * 512
        x4 = jnp.concatenate([x[:, c0 + g * 128:c0 + (g + 1) * 128]
                              for g in range(4)], axis=0)
        x4t = lax.dot_general(x4, eye, (((0,), (0,)), ((), ())),
                              preferred_element_type=jnp.float32)
        out_ref[pl.ds(m * 128, 128), :] = x4t


def _pack_tc(tab_t):
    emb, v = tab_t.shape
    nt = -(-v // _PACK)
    eye = jnp.eye(4 * emb, dtype=jnp.float32)
    return pl.pallas_call(
        _pack_body,
        grid=(nt,),
        in_specs=[
            pl.BlockSpec((4 * emb, 4 * emb), lambda j: (0, 0)),
            pl.BlockSpec((emb, _PACK), lambda j: (0, j)),
        ],
        out_specs=pl.BlockSpec((_PACK // 4, 4 * emb), lambda j: (j, 0)),
        out_shape=jax.ShapeDtypeStruct((nt * (_PACK // 4), 4 * emb),
                                       jnp.float32),
    )(eye, tab_t)


def _make_sc_gather(emb4, batch, nc, ns):
    nw = nc * ns
    b_per_w = batch // nw
    n_chunks = b_per_w // _IDX_CHUNK
    mesh = plsc.VectorSubcoreMesh(core_axis_name="c", subcore_axis_name="s")

    @functools.partial(
        pl.kernel,
        mesh=mesh,
        out_type=[
            jax.ShapeDtypeStruct((batch, emb4), jnp.float32),
            jax.ShapeDtypeStruct((batch, emb4), jnp.float32),
        ],
        scratch_types=[
            pltpu.VMEM((8, _IDX_CHUNK), jnp.int32),
            pltpu.VMEM((_IDX_CHUNK, emb4), jnp.float32),
            pltpu.VMEM((8, _IDX_CHUNK), jnp.int32),
            pltpu.VMEM((_IDX_CHUNK, emb4), jnp.float32),
            pltpu.SemaphoreType.DMA,
            pltpu.SemaphoreType.DMA,
        ],
    )
    def gather_k(uid_hbm, utab_hbm, iid_hbm, itab_hbm, ue_hbm, ie_hbm,
                 uidx_v, urows_v, iidx_v, irows_v, usem, isem):
        wid = lax.axis_index("s") * nc + lax.axis_index("c")
        base = wid * b_per_w
        for j in range(n_chunks):
            pltpu.sync_copy(uid_hbm.at[pl.ds(base + j * _IDX_CHUNK, _IDX_CHUNK)],
                            uidx_v.at[j])
            pltpu.sync_copy(iid_hbm.at[pl.ds(base + j * _IDX_CHUNK, _IDX_CHUNK)],
                            iidx_v.at[j])
        for h in range(n_chunks):
            ucp = pltpu.async_copy(utab_hbm.at[uidx_v.at[h]], urows_v, usem)
            icp = pltpu.async_copy(itab_hbm.at[iidx_v.at[h]], irows_v, isem)
            ucp.wait()
            icp.wait()
            pltpu.sync_copy(urows_v,
                            ue_hbm.at[pl.ds(base + h * _IDX_CHUNK, _IDX_CHUNK)])
            pltpu.sync_copy(irows_v,
                            ie_hbm.at[pl.ds(base + h * _IDX_CHUNK, _IDX_CHUNK)])

    return gather_k


def _quarter_select(x, sel_ref, emb):
    acc = None
    for k in range(4):
        part = jnp.where(sel_ref[...] == k, x[:, k * emb:(k + 1) * emb], 0.0)
        acc = part if acc is None else acc + part
    return acc


def _mlp_body(ue_ref, ie_ref, uq_ref, iq_ref, w1u_ref, w1i_ref, b1_ref,
              w2_ref, b2_ref, w3_ref, b3_ref, out_ref):
    emb = ue_ref.shape[1] // 4
    ue = _quarter_select(ue_ref[...], uq_ref, emb)
    ie = _quarter_select(ie_ref[...], iq_ref, emb)
    cdims = (((1,), (1,)), ((), ()))
    h1 = lax.dot_general(ue, w1u_ref[...], cdims,
                         preferred_element_type=jnp.float32)
    h1 = h1 + lax.dot_general(ie, w1i_ref[...], cdims,
                              preferred_element_type=jnp.float32)
    h1 = jnp.maximum(h1 + b1_ref[...], 0.0)
    h2 = lax.dot_general(h1, w2_ref[...], cdims,
                         preferred_element_type=jnp.float32)
    h2 = jnp.maximum(h2 + b2_ref[...], 0.0)
    logit = jnp.sum(h2 * w3_ref[...], axis=1, keepdims=True) + b3_ref[...]
    out_ref[...] = jax.nn.sigmoid(logit)


def _mlp_tc(ue2, ie2, uq, iq, W1u, W1i, b1, W2, b2, W3, b3, block_b):
    batch = ue2.shape[0]
    grid = (batch // block_b,)
    full = lambda shape: pl.BlockSpec(shape, lambda i: (0, 0))
    return pl.pallas_call(
        _mlp_body,
        grid=grid,
        in_specs=[
            pl.BlockSpec((block_b, ue2.shape[1]), lambda i: (i, 0)),
            pl.BlockSpec((block_b, ie2.shape[1]), lambda i: (i, 0)),
            pl.BlockSpec((block_b, 1), lambda i: (i, 0)),
            pl.BlockSpec((block_b, 1), lambda i: (i, 0)),
            full(W1u.shape),
            full(W1i.shape),
            full(b1.shape),
            full(W2.shape),
            full(b2.shape),
            full(W3.shape),
            full(b3.shape),
        ],
        out_specs=pl.BlockSpec((block_b, 1), lambda i: (i, 0)),
        out_shape=jax.ShapeDtypeStruct((batch, 1), jnp.float32),
    )(ue2, ie2, uq, iq, W1u, W1i, b1, W2, b2, W3, b3)


def kernel(user_ids, item_ids, user_table, item_table, W1, b1, W2, b2, W3, b3):
    batch = user_ids.shape[0]
    emb = user_table.shape[1]

    uid = user_ids.astype(jnp.int32)
    iid = item_ids.astype(jnp.int32)
    upidx = (uid >> 9) * 128 + (uid & 127)
    ipidx = (iid >> 9) * 128 + (iid & 127)
    uq = ((uid >> 7) & 3).reshape(batch, 1)
    iq = ((iid >> 7) & 3).reshape(batch, 1)

    ut_p = _pack_tc(user_table.T)
    it_p = _pack_tc(item_table.T)

    info = plsc.get_sparse_core_info()
    gather_k = _make_sc_gather(4 * emb, batch, info.num_cores,
                               info.num_subcores)
    ue2, ie2 = gather_k(upidx, ut_p, ipidx, it_p)

    W1u = W1[:, :emb]
    W1i = W1[:, emb:]
    return _mlp_tc(ue2, ie2, uq, iq, W1u, W1i, b1.reshape(1, -1), W2,
                   b2.reshape(1, -1), W3, b3.reshape(1, 1), block_b=2048)


# speedup vs baseline: 4.6308x; 4.6308x over previous
import functools

import jax
import jax.numpy as jnp
from jax import lax
from jax.experimental import pallas as pl
from jax.experimental.pallas import tpu as pltpu
from jax.experimental.pallas import tpu_sc as plsc

_IDX_CHUNK = 128
_PACK = 8192


def _pack_body(eye_ref, in_ref, out_ref):
    x = in_ref[...]
    eye = eye_ref[...]
    for m in range(_PACK // 512):
        c0 = m * 512
        x4 = jnp.concatenate([x[:, c0 + g * 128:c0 + (g + 1) * 128]
                              for g in range(4)], axis=0)
        x4t = lax.dot_general(x4, eye, (((0,), (0,)), ((), ())),
                              preferred_element_type=jnp.float32)
        out_ref[pl.ds(m * 128, 128), :] = x4t


def _pack_tc(tab_t):
    emb, v = tab_t.shape
    nt = -(-v // _PACK)
    eye = jnp.eye(4 * emb, dtype=jnp.float32)
    return pl.pallas_call(
        _pack_body,
        grid=(nt,),
        in_specs=[
            pl.BlockSpec((4 * emb, 4 * emb), lambda j: (0, 0)),
            pl.BlockSpec((emb, _PACK), lambda j: (0, j)),
        ],
        out_specs=pl.BlockSpec((_PACK // 4, 4 * emb), lambda j: (j, 0)),
        out_shape=jax.ShapeDtypeStruct((nt * (_PACK // 4), 4 * emb),
                                       jnp.float32),
    )(eye, tab_t)


def _make_sc_gather(emb4, batch, nc, ns):
    nw = nc * ns
    b_per_w = batch // nw
    n_chunks = b_per_w // _IDX_CHUNK
    mesh = plsc.VectorSubcoreMesh(core_axis_name="c", subcore_axis_name="s")

    @functools.partial(
        pl.kernel,
        mesh=mesh,
        out_type=[
            jax.ShapeDtypeStruct((batch, emb4), jnp.float32),
            jax.ShapeDtypeStruct((batch, emb4), jnp.float32),
        ],
        scratch_types=[
            pltpu.VMEM((8, _IDX_CHUNK), jnp.int32),
            pltpu.VMEM((_IDX_CHUNK, emb4), jnp.float32),
            pltpu.VMEM((8, _IDX_CHUNK), jnp.int32),
            pltpu.VMEM((_IDX_CHUNK, emb4), jnp.float32),
            pltpu.SemaphoreType.DMA,
            pltpu.SemaphoreType.DMA,
        ],
    )
    def gather_k(uid_hbm, utab_hbm, iid_hbm, itab_hbm, ue_hbm, ie_hbm,
                 uidx_v, urows_v, iidx_v, irows_v, usem, isem):
        wid = lax.axis_index("s") * nc + lax.axis_index("c")
        base = wid * b_per_w
        for j in range(n_chunks):
            pltpu.sync_copy(uid_hbm.at[pl.ds(base + j * _IDX_CHUNK, _IDX_CHUNK)],
                            uidx_v.at[j])
            pltpu.sync_copy(iid_hbm.at[pl.ds(base + j * _IDX_CHUNK, _IDX_CHUNK)],
                            iidx_v.at[j])
        for h in range(n_chunks):
            ucp = pltpu.async_copy(utab_hbm.at[uidx_v.at[h]], urows_v, usem)
            icp = pltpu.async_copy(itab_hbm.at[iidx_v.at[h]], irows_v, isem)
            ucp.wait()
            icp.wait()
            pltpu.sync_copy(urows_v,
                            ue_hbm.at[pl.ds(base + h * _IDX_CHUNK, _IDX_CHUNK)])
            pltpu.sync_copy(irows_v,
                            ie_hbm.at[pl.ds(base + h * _IDX_CHUNK, _IDX_CHUNK)])

    return gather_k


def _quarter_select(x, sel_ref, emb):
    acc = None
    for k in range(4):
        part = jnp.where(sel_ref[...] == k, x[:, k * emb:(k + 1) * emb], 0.0)
        acc = part if acc is None else acc + part
    return acc


def _mlp_body(ue_ref, ie_ref, uq_ref, iq_ref, w1u_ref, w1i_ref, b1_ref,
              w2_ref, b2_ref, w3_ref, b3_ref, out_ref):
    emb = ue_ref.shape[1] // 4
    ue = _quarter_select(ue_ref[...], uq_ref, emb)
    ie = _quarter_select(ie_ref[...], iq_ref, emb)
    cdims = (((1,), (1,)), ((), ()))
    h1 = lax.dot_general(ue, w1u_ref[...], cdims,
                         preferred_element_type=jnp.float32)
    h1 = h1 + lax.dot_general(ie, w1i_ref[...], cdims,
                              preferred_element_type=jnp.float32)
    h1 = jnp.maximum(h1 + b1_ref[...], 0.0)
    h2 = lax.dot_general(h1, w2_ref[...], cdims,
                         preferred_element_type=jnp.float32)
    h2 = jnp.maximum(h2 + b2_ref[...], 0.0)
    logit = jnp.sum(h2 * w3_ref[...], axis=1, keepdims=True) + b3_ref[...]
    out_ref[...] = jax.nn.sigmoid(logit)


def _mlp_tc(ue2, ie2, uq, iq, W1u, W1i, b1, W2, b2, W3, b3, block_b):
    batch = ue2.shape[0]
    grid = (batch // block_b,)
    full = lambda shape: pl.BlockSpec(shape, lambda i: (0, 0))
    return pl.pallas_call(
        _mlp_body,
        grid=grid,
        in_specs=[
            pl.BlockSpec((block_b, ue2.shape[1]), lambda i: (i, 0)),
            pl.BlockSpec((block_b, ie2.shape[1]), lambda i: (i, 0)),
            pl.BlockSpec((block_b, 1), lambda i: (i, 0)),
            pl.BlockSpec((block_b, 1), lambda i: (i, 0)),
            full(W1u.shape),
            full(W1i.shape),
            full(b1.shape),
            full(W2.shape),
            full(b2.shape),
            full(W3.shape),
            full(b3.shape),
        ],
        out_specs=pl.BlockSpec((block_b, 1), lambda i: (i, 0)),
        out_shape=jax.ShapeDtypeStruct((batch, 1), jnp.float32),
    )(ue2, ie2, uq, iq, W1u, W1i, b1, W2, b2, W3, b3)


def kernel(user_ids, item_ids, user_table, item_table, W1, b1, W2, b2, W3, b3):
    batch = user_ids.shape[0]
    emb = user_table.shape[1]

    uid = user_ids.astype(jnp.int32)
    iid = item_ids.astype(jnp.int32)
    upidx = (uid >> 9) * 128 + (uid & 127)
    ipidx = (iid >> 9) * 128 + (iid & 127)
    uq = ((uid >> 7) & 3).reshape(batch, 1)
    iq = ((iid >> 7) & 3).reshape(batch, 1)

    ut_p = _pack_tc(user_table.T)
    it_p = _pack_tc(item_table.T)

    info = plsc.get_sparse_core_info()
    gather_k = _make_sc_gather(4 * emb, batch, info.num_cores,
                               info.num_subcores)
    ue2, ie2 = gather_k(upidx, ut_p, ipidx, it_p)

    W1u = W1[:, :emb]
    W1i = W1[:, emb:]
    return _mlp_tc(ue2, ie2, uq, iq, W1u, W1i, b1.reshape(1, -1), W2,
                   b2.reshape(1, -1), W3, b3.reshape(1, 1), block_b=2048)


# speedup vs baseline: 5.1021x; 1.1018x over previous
import functools

import jax
import jax.numpy as jnp
from jax import lax
from jax.experimental import pallas as pl
from jax.experimental.pallas import tpu as pltpu
from jax.experimental.pallas import tpu_sc as plsc

_IDX_CHUNK = 128
_PACK = 16384


def _pack_body(eye_ref, in_ref, out_ref):
    x = in_ref[...]
    eye = eye_ref[...]
    for m in range(_PACK // 512):
        c0 = m * 512
        x4 = jnp.concatenate([x[:, c0 + g * 128:c0 + (g + 1) * 128]
                              for g in range(4)], axis=0)
        x4t = lax.dot_general(x4, eye, (((0,), (0,)), ((), ())),
                              preferred_element_type=jnp.float32)
        out_ref[pl.ds(m * 128, 128), :] = x4t


def _pack_tc(tab_t):
    emb, v = tab_t.shape
    nt = -(-v // _PACK)
    eye = jnp.eye(4 * emb, dtype=jnp.float32)
    return pl.pallas_call(
        _pack_body,
        grid=(nt,),
        in_specs=[
            pl.BlockSpec((4 * emb, 4 * emb), lambda j: (0, 0)),
            pl.BlockSpec((emb, _PACK), lambda j: (0, j)),
        ],
        out_specs=pl.BlockSpec((_PACK // 4, 4 * emb), lambda j: (j, 0)),
        out_shape=jax.ShapeDtypeStruct((nt * (_PACK // 4), 4 * emb),
                                       jnp.float32),
    )(eye, tab_t)


def _make_sc_gather(emb4, batch, nc, ns):
    nw = nc * ns
    b_per_w = batch // nw
    n_chunks = b_per_w // _IDX_CHUNK
    mesh = plsc.VectorSubcoreMesh(core_axis_name="c", subcore_axis_name="s")

    @functools.partial(
        pl.kernel,
        mesh=mesh,
        out_type=[
            jax.ShapeDtypeStruct((batch, emb4), jnp.float32),
            jax.ShapeDtypeStruct((batch, emb4), jnp.float32),
        ],
        scratch_types=[
            pltpu.VMEM((8, _IDX_CHUNK), jnp.int32),
            pltpu.VMEM((_IDX_CHUNK, emb4), jnp.float32),
            pltpu.VMEM((8, _IDX_CHUNK), jnp.int32),
            pltpu.VMEM((_IDX_CHUNK, emb4), jnp.float32),
            pltpu.SemaphoreType.DMA,
            pltpu.SemaphoreType.DMA,
        ],
    )
    def gather_k(uid_hbm, utab_hbm, iid_hbm, itab_hbm, ue_hbm, ie_hbm,
                 uidx_v, urows_v, iidx_v, irows_v, usem, isem):
        wid = lax.axis_index("s") * nc + lax.axis_index("c")
        base = wid * b_per_w
        for j in range(n_chunks):
            pltpu.sync_copy(uid_hbm.at[pl.ds(base + j * _IDX_CHUNK, _IDX_CHUNK)],
                            uidx_v.at[j])
            pltpu.sync_copy(iid_hbm.at[pl.ds(base + j * _IDX_CHUNK, _IDX_CHUNK)],
                            iidx_v.at[j])
        for h in range(n_chunks):
            ucp = pltpu.async_copy(utab_hbm.at[uidx_v.at[h]], urows_v, usem)
            icp = pltpu.async_copy(itab_hbm.at[iidx_v.at[h]], irows_v, isem)
            ucp.wait()
            icp.wait()
            pltpu.sync_copy(urows_v,
                            ue_hbm.at[pl.ds(base + h * _IDX_CHUNK, _IDX_CHUNK)])
            pltpu.sync_copy(irows_v,
                            ie_hbm.at[pl.ds(base + h * _IDX_CHUNK, _IDX_CHUNK)])

    return gather_k


def _quarter_select(x, sel_ref, emb):
    acc = None
    for k in range(4):
        part = jnp.where(sel_ref[...] == k, x[:, k * emb:(k + 1) * emb], 0.0)
        acc = part if acc is None else acc + part
    return acc


def _mlp_body(ue_ref, ie_ref, uq_ref, iq_ref, w1u_ref, w1i_ref, b1_ref,
              w2_ref, b2_ref, w3_ref, b3_ref, out_ref):
    emb = ue_ref.shape[1] // 4
    ue = _quarter_select(ue_ref[...], uq_ref, emb)
    ie = _quarter_select(ie_ref[...], iq_ref, emb)
    cdims = (((1,), (1,)), ((), ()))
    h1 = lax.dot_general(ue, w1u_ref[...], cdims,
                         preferred_element_type=jnp.float32)
    h1 = h1 + lax.dot_general(ie, w1i_ref[...], cdims,
                              preferred_element_type=jnp.float32)
    h1 = jnp.maximum(h1 + b1_ref[...], 0.0)
    h2 = lax.dot_general(h1, w2_ref[...], cdims,
                         preferred_element_type=jnp.float32)
    h2 = jnp.maximum(h2 + b2_ref[...], 0.0)
    logit = jnp.sum(h2 * w3_ref[...], axis=1, keepdims=True) + b3_ref[...]
    out_ref[...] = jax.nn.sigmoid(logit)


def _mlp_tc(ue2, ie2, uq, iq, W1u, W1i, b1, W2, b2, W3, b3, block_b):
    batch = ue2.shape[0]
    grid = (batch // block_b,)
    full = lambda shape: pl.BlockSpec(shape, lambda i: (0, 0))
    return pl.pallas_call(
        _mlp_body,
        grid=grid,
        in_specs=[
            pl.BlockSpec((block_b, ue2.shape[1]), lambda i: (i, 0)),
            pl.BlockSpec((block_b, ie2.shape[1]), lambda i: (i, 0)),
            pl.BlockSpec((block_b, 1), lambda i: (i, 0)),
            pl.BlockSpec((block_b, 1), lambda i: (i, 0)),
            full(W1u.shape),
            full(W1i.shape),
            full(b1.shape),
            full(W2.shape),
            full(b2.shape),
            full(W3.shape),
            full(b3.shape),
        ],
        out_specs=pl.BlockSpec((block_b, 1), lambda i: (i, 0)),
        out_shape=jax.ShapeDtypeStruct((batch, 1), jnp.float32),
    )(ue2, ie2, uq, iq, W1u, W1i, b1, W2, b2, W3, b3)


def kernel(user_ids, item_ids, user_table, item_table, W1, b1, W2, b2, W3, b3):
    batch = user_ids.shape[0]
    emb = user_table.shape[1]

    uid = user_ids.astype(jnp.int32)
    iid = item_ids.astype(jnp.int32)
    upidx = (uid >> 9) * 128 + (uid & 127)
    ipidx = (iid >> 9) * 128 + (iid & 127)
    uq = ((uid >> 7) & 3).reshape(batch, 1)
    iq = ((iid >> 7) & 3).reshape(batch, 1)

    ut_p = _pack_tc(user_table.T)
    it_p = _pack_tc(item_table.T)

    info = plsc.get_sparse_core_info()
    gather_k = _make_sc_gather(4 * emb, batch, info.num_cores,
                               info.num_subcores)
    ue2, ie2 = gather_k(upidx, ut_p, ipidx, it_p)

    W1u = W1[:, :emb]
    W1i = W1[:, emb:]
    return _mlp_tc(ue2, ie2, uq, iq, W1u, W1i, b1.reshape(1, -1), W2,
                   b2.reshape(1, -1), W3, b3.reshape(1, 1), block_b=2048)


# speedup vs baseline: 5.2072x; 1.0206x over previous
import functools

import jax
import jax.numpy as jnp
from jax import lax
from jax.experimental import pallas as pl
from jax.experimental.pallas import tpu as pltpu
from jax.experimental.pallas import tpu_sc as plsc

_IDX_CHUNK = 128
_PACK = 32768


def _pack_body(eye_ref, in_ref, out_ref):
    x = in_ref[...]
    eye = eye_ref[...]
    for m in range(_PACK // 512):
        c0 = m * 512
        x4 = jnp.concatenate([x[:, c0 + g * 128:c0 + (g + 1) * 128]
                              for g in range(4)], axis=0)
        x4t = lax.dot_general(x4, eye, (((0,), (0,)), ((), ())),
                              preferred_element_type=jnp.float32)
        out_ref[pl.ds(m * 128, 128), :] = x4t


def _pack_tc(tab_t):
    emb, v = tab_t.shape
    nt = -(-v // _PACK)
    eye = jnp.eye(4 * emb, dtype=jnp.float32)
    return pl.pallas_call(
        _pack_body,
        grid=(nt,),
        in_specs=[
            pl.BlockSpec((4 * emb, 4 * emb), lambda j: (0, 0)),
            pl.BlockSpec((emb, _PACK), lambda j: (0, j)),
        ],
        out_specs=pl.BlockSpec((_PACK // 4, 4 * emb), lambda j: (j, 0)),
        out_shape=jax.ShapeDtypeStruct((nt * (_PACK // 4), 4 * emb),
                                       jnp.float32),
    )(eye, tab_t)


def _make_sc_gather(emb4, batch, nc, ns):
    nw = nc * ns
    b_per_w = batch // nw
    n_chunks = b_per_w // _IDX_CHUNK
    mesh = plsc.VectorSubcoreMesh(core_axis_name="c", subcore_axis_name="s")

    @functools.partial(
        pl.kernel,
        mesh=mesh,
        out_type=[
            jax.ShapeDtypeStruct((batch, emb4), jnp.float32),
            jax.ShapeDtypeStruct((batch, emb4), jnp.float32),
        ],
        scratch_types=[
            pltpu.VMEM((8, _IDX_CHUNK), jnp.int32),
            pltpu.VMEM((_IDX_CHUNK, emb4), jnp.float32),
            pltpu.VMEM((8, _IDX_CHUNK), jnp.int32),
            pltpu.VMEM((_IDX_CHUNK, emb4), jnp.float32),
            pltpu.SemaphoreType.DMA,
            pltpu.SemaphoreType.DMA,
        ],
    )
    def gather_k(uid_hbm, utab_hbm, iid_hbm, itab_hbm, ue_hbm, ie_hbm,
                 uidx_v, urows_v, iidx_v, irows_v, usem, isem):
        wid = lax.axis_index("s") * nc + lax.axis_index("c")
        base = wid * b_per_w
        for j in range(n_chunks):
            pltpu.sync_copy(uid_hbm.at[pl.ds(base + j * _IDX_CHUNK, _IDX_CHUNK)],
                            uidx_v.at[j])
            pltpu.sync_copy(iid_hbm.at[pl.ds(base + j * _IDX_CHUNK, _IDX_CHUNK)],
                            iidx_v.at[j])
        for h in range(n_chunks):
            ucp = pltpu.async_copy(utab_hbm.at[uidx_v.at[h]], urows_v, usem)
            icp = pltpu.async_copy(itab_hbm.at[iidx_v.at[h]], irows_v, isem)
            ucp.wait()
            icp.wait()
            pltpu.sync_copy(urows_v,
                            ue_hbm.at[pl.ds(base + h * _IDX_CHUNK, _IDX_CHUNK)])
            pltpu.sync_copy(irows_v,
                            ie_hbm.at[pl.ds(base + h * _IDX_CHUNK, _IDX_CHUNK)])

    return gather_k


def _quarter_select(x, sel_ref, emb):
    acc = None
    for k in range(4):
        part = jnp.where(sel_ref[...] == k, x[:, k * emb:(k + 1) * emb], 0.0)
        acc = part if acc is None else acc + part
    return acc


def _mlp_body(ue_ref, ie_ref, uq_ref, iq_ref, w1u_ref, w1i_ref, b1_ref,
              w2_ref, b2_ref, w3_ref, b3_ref, out_ref):
    emb = ue_ref.shape[1] // 4
    ue = _quarter_select(ue_ref[...], uq_ref, emb)
    ie = _quarter_select(ie_ref[...], iq_ref, emb)
    cdims = (((1,), (1,)), ((), ()))
    h1 = lax.dot_general(ue, w1u_ref[...], cdims,
                         preferred_element_type=jnp.float32)
    h1 = h1 + lax.dot_general(ie, w1i_ref[...], cdims,
                              preferred_element_type=jnp.float32)
    h1 = jnp.maximum(h1 + b1_ref[...], 0.0)
    h2 = lax.dot_general(h1, w2_ref[...], cdims,
                         preferred_element_type=jnp.float32)
    h2 = jnp.maximum(h2 + b2_ref[...], 0.0)
    logit = jnp.sum(h2 * w3_ref[...], axis=1, keepdims=True) + b3_ref[...]
    out_ref[...] = jax.nn.sigmoid(logit)


def _mlp_tc(ue2, ie2, uq, iq, W1u, W1i, b1, W2, b2, W3, b3, block_b):
    batch = ue2.shape[0]
    grid = (batch // block_b,)
    full = lambda shape: pl.BlockSpec(shape, lambda i: (0, 0))
    return pl.pallas_call(
        _mlp_body,
        grid=grid,
        in_specs=[
            pl.BlockSpec((block_b, ue2.shape[1]), lambda i: (i, 0)),
            pl.BlockSpec((block_b, ie2.shape[1]), lambda i: (i, 0)),
            pl.BlockSpec((block_b, 1), lambda i: (i, 0)),
            pl.BlockSpec((block_b, 1), lambda i: (i, 0)),
            full(W1u.shape),
            full(W1i.shape),
            full(b1.shape),
            full(W2.shape),
            full(b2.shape),
            full(W3.shape),
            full(b3.shape),
        ],
        out_specs=pl.BlockSpec((block_b, 1), lambda i: (i, 0)),
        out_shape=jax.ShapeDtypeStruct((batch, 1), jnp.float32),
    )(ue2, ie2, uq, iq, W1u, W1i, b1, W2, b2, W3, b3)


def kernel(user_ids, item_ids, user_table, item_table, W1, b1, W2, b2, W3, b3):
    batch = user_ids.shape[0]
    emb = user_table.shape[1]

    uid = user_ids.astype(jnp.int32)
    iid = item_ids.astype(jnp.int32)
    upidx = (uid >> 9) * 128 + (uid & 127)
    ipidx = (iid >> 9) * 128 + (iid & 127)
    uq = ((uid >> 7) & 3).reshape(batch, 1)
    iq = ((iid >> 7) & 3).reshape(batch, 1)

    ut_p = _pack_tc(user_table.T)
    it_p = _pack_tc(item_table.T)

    info = plsc.get_sparse_core_info()
    gather_k = _make_sc_gather(4 * emb, batch, info.num_cores,
                               info.num_subcores)
    ue2, ie2 = gather_k(upidx, ut_p, ipidx, it_p)

    W1u = W1[:, :emb]
    W1i = W1[:, emb:]
    return _mlp_tc(ue2, ie2, uq, iq, W1u, W1i, b1.reshape(1, -1), W2,
                   b2.reshape(1, -1), W3, b3.reshape(1, 1), block_b=2048)


# speedup vs baseline: 5.2787x; 1.0137x over previous
import functools

import jax
import jax.numpy as jnp
from jax import lax
from jax.experimental import pallas as pl
from jax.experimental.pallas import tpu as pltpu
from jax.experimental.pallas import tpu_sc as plsc

_IDX_CHUNK = 128
_PACK = 32768


def _pack_body(eye_ref, in_ref, out_ref):
    x = in_ref[...]
    eye = eye_ref[...]
    for m in range(_PACK // 512):
        c0 = m * 512
        x4 = jnp.concatenate([x[:, c0 + g * 128:c0 + (g + 1) * 128]
                              for g in range(4)], axis=0)
        x4t = lax.dot_general(x4, eye, (((0,), (0,)), ((), ())),
                              preferred_element_type=jnp.float32)
        out_ref[pl.ds(m * 128, 128), :] = x4t


def _pack_tc(tab_t):
    emb, v = tab_t.shape
    nt = -(-v // _PACK)
    eye = jnp.eye(4 * emb, dtype=jnp.float32)
    return pl.pallas_call(
        _pack_body,
        grid=(nt,),
        in_specs=[
            pl.BlockSpec((4 * emb, 4 * emb), lambda j: (0, 0)),
            pl.BlockSpec((emb, _PACK), lambda j: (0, j)),
        ],
        out_specs=pl.BlockSpec((_PACK // 4, 4 * emb), lambda j: (j, 0)),
        out_shape=jax.ShapeDtypeStruct((nt * (_PACK // 4), 4 * emb),
                                       jnp.float32),
    )(eye, tab_t)


def _make_sc_gather(emb4, batch, nc, ns):
    nw = nc * ns
    b_per_w = batch // nw
    n_chunks = b_per_w // _IDX_CHUNK
    mesh = plsc.VectorSubcoreMesh(core_axis_name="c", subcore_axis_name="s")

    @functools.partial(
        pl.kernel,
        mesh=mesh,
        out_type=jax.ShapeDtypeStruct((batch, emb4), jnp.float32),
        scratch_types=[
            pltpu.VMEM((8, _IDX_CHUNK), jnp.int32),
            pltpu.VMEM((8, _IDX_CHUNK), jnp.int32),
            pltpu.VMEM((_IDX_CHUNK, emb4), jnp.float32),
            pltpu.VMEM((_IDX_CHUNK, emb4), jnp.float32),
            pltpu.SemaphoreType.DMA,
        ],
    )
    def gather_k(ids_hbm, tab_hbm, out_hbm, raw_v, idx_v, rows_a, rows_b, sem):
        wid = lax.axis_index("s") * nc + lax.axis_index("c")
        base = wid * b_per_w
        for j in range(n_chunks):
            pltpu.sync_copy(ids_hbm.at[pl.ds(base + j * _IDX_CHUNK, _IDX_CHUNK)],
                            raw_v.at[j])
        for j in range(n_chunks):
            for v in range(_IDX_CHUNK // 16):
                x = raw_v[j, pl.ds(v * 16, 16)]
                idx_v[j, pl.ds(v * 16, 16)] = (x >> 9) * 128 + (x & 127)
        bufs = (rows_a, rows_b)
        copies = [None, None]
        for h in range(n_chunks):
            copies[h % 2] = pltpu.async_copy(tab_hbm.at[idx_v.at[h]],
                                             bufs[h % 2], sem)
            if h > 0:
                copies[(h - 1) % 2].wait()
                pltpu.sync_copy(bufs[(h - 1) % 2],
                                out_hbm.at[pl.ds(base + (h - 1) * _IDX_CHUNK,
                                                 _IDX_CHUNK)])
        copies[(n_chunks - 1) % 2].wait()
        pltpu.sync_copy(bufs[(n_chunks - 1) % 2],
                        out_hbm.at[pl.ds(base + (n_chunks - 1) * _IDX_CHUNK,
                                         _IDX_CHUNK)])

    return gather_k


def _quarter_select(x, sel, emb):
    acc = None
    for k in range(4):
        part = jnp.where(sel == k, x[:, k * emb:(k + 1) * emb], 0.0)
        acc = part if acc is None else acc + part
    return acc


def _mlp_body(ue_ref, ie_ref, uid_ref, iid_ref, w1u_ref, w1i_ref, b1_ref,
              w2_ref, b2_ref, w3_ref, b3_ref, out_ref):
    emb = ue_ref.shape[1] // 4
    uq = (uid_ref[...] >> 7) & 3
    iq = (iid_ref[...] >> 7) & 3
    ue = _quarter_select(ue_ref[...], uq, emb)
    ie = _quarter_select(ie_ref[...], iq, emb)
    cdims = (((1,), (1,)), ((), ()))
    h1 = lax.dot_general(ue, w1u_ref[...], cdims,
                         preferred_element_type=jnp.float32)
    h1 = h1 + lax.dot_general(ie, w1i_ref[...], cdims,
                              preferred_element_type=jnp.float32)
    h1 = jnp.maximum(h1 + b1_ref[...], 0.0)
    h2 = lax.dot_general(h1, w2_ref[...], cdims,
                         preferred_element_type=jnp.float32)
    h2 = jnp.maximum(h2 + b2_ref[...], 0.0)
    logit = jnp.sum(h2 * w3_ref[...], axis=1, keepdims=True) + b3_ref[...]
    out_ref[...] = jax.nn.sigmoid(logit)


def _mlp_tc(ue2, ie2, uq, iq, W1u, W1i, b1, W2, b2, W3, b3, block_b):
    batch = ue2.shape[0]
    grid = (batch // block_b,)
    full = lambda shape: pl.BlockSpec(shape, lambda i: (0, 0))
    return pl.pallas_call(
        _mlp_body,
        grid=grid,
        in_specs=[
            pl.BlockSpec((block_b, ue2.shape[1]), lambda i: (i, 0)),
            pl.BlockSpec((block_b, ie2.shape[1]), lambda i: (i, 0)),
            pl.BlockSpec((block_b, 1), lambda i: (i, 0)),
            pl.BlockSpec((block_b, 1), lambda i: (i, 0)),
            full(W1u.shape),
            full(W1i.shape),
            full(b1.shape),
            full(W2.shape),
            full(b2.shape),
            full(W3.shape),
            full(b3.shape),
        ],
        out_specs=pl.BlockSpec((block_b, 1), lambda i: (i, 0)),
        out_shape=jax.ShapeDtypeStruct((batch, 1), jnp.float32),
    )(ue2, ie2, uq, iq, W1u, W1i, b1, W2, b2, W3, b3)


def kernel(user_ids, item_ids, user_table, item_table, W1, b1, W2, b2, W3, b3):
    batch = user_ids.shape[0]
    emb = user_table.shape[1]

    uid = user_ids.astype(jnp.int32)
    iid = item_ids.astype(jnp.int32)

    info = plsc.get_sparse_core_info()
    gather_k = _make_sc_gather(4 * emb, batch, info.num_cores,
                               info.num_subcores)

    ut_p = _pack_tc(user_table.T)
    ue2 = gather_k(uid, ut_p)
    it_p = _pack_tc(item_table.T)
    ie2 = gather_k(iid, it_p)

    W1u = W1[:, :emb]
    W1i = W1[:, emb:]
    return _mlp_tc(ue2, ie2, uid.reshape(batch, 1), iid.reshape(batch, 1),
                   W1u, W1i, b1.reshape(1, -1), W2, b2.reshape(1, -1), W3,
                   b3.reshape(1, 1), block_b=4096)
